# initial kernel scaffold (unmeasured)
import jax
import jax.numpy as jnp
from jax import lax
from jax.experimental import pallas as pl
from jax.experimental.pallas import tpu as pltpu

N_DEV = 4
HQ = 32
HQ_PER = 8
DH = 128
SQ = 256
SKV = 4096
DMODEL = 1024
QBLK = 64
NQB = SQ // QBLK
NGRP = SKV // (4 * QBLK)
SCALE = 0.08838834764831843
BF16 = jnp.bfloat16
F32 = jnp.float32


def kernel(x, Wq, K_ext, V_ext, Wo):
    x2 = x.reshape(SQ, DMODEL)
    Kr = K_ext.reshape(NGRP, 4, QBLK, HQ, DH)
    Vr = V_ext.reshape(NGRP, 4, QBLK, HQ, DH)

    def body(x_ref, wq_ref, k_hbm, v_hbm, wo_ref, out_ref,
             k_buf, v_buf, ctx_ref, comm_ref, copy_sems, send_sems, recv_sems):
        my = lax.axis_index("i")
        left = lax.rem(my + N_DEV - 1, N_DEV)
        right = lax.rem(my + 1, N_DEV)
        h0 = my * HQ_PER

        barrier = pltpu.get_barrier_semaphore()
        for nbr in (left, right):
            pl.semaphore_signal(barrier, inc=1, device_id=(nbr,),
                                device_id_type=pl.DeviceIdType.MESH)

        copies = []
        for h in range(HQ_PER):
            ck = pltpu.make_async_copy(
                k_hbm.at[:, :, :, h0 + h, :], k_buf.at[h], copy_sems.at[0])
            cv = pltpu.make_async_copy(
                v_hbm.at[:, :, :, h0 + h, :], v_buf.at[h], copy_sems.at[1])
            ck.start()
            cv.start()
            copies += [ck, cv]

        pl.semaphore_wait(barrier, 2)

        q = lax.dot(x_ref[...].astype(BF16), wq_ref[...].astype(BF16),
                    preferred_element_type=F32)

        for c in copies:
            c.wait()

        for h in range(HQ_PER):
            for qb in range(NQB):
                q_blk = q[qb * QBLK:(qb + 1) * QBLK,
                          h * DH:(h + 1) * DH].astype(BF16)
                k_sel = k_buf[h, :, qb, :, :].reshape(NGRP * QBLK, DH)
                s = lax.dot_general(
                    q_blk, k_sel.astype(BF16), (((1,), (1,)), ((), ())),
                    preferred_element_type=F32) * SCALE
                m = jnp.max(s, axis=-1, keepdims=True)
                w = jnp.exp(s - m)
                p = (w / jnp.sum(w, axis=-1, keepdims=True)).astype(BF16)
                v_sel = v_buf[h, :, qb, :, :].reshape(NGRP * QBLK, DH)
                ctx_blk = lax.dot_general(
                    p, v_sel.astype(BF16), (((1,), (0,)), ((), ())),
                    preferred_element_type=F32)
                ctx_ref[qb * QBLK:(qb + 1) * QBLK,
                        h * DH:(h + 1) * DH] = ctx_blk

        partial = lax.dot(ctx_ref[...].astype(BF16), wo_ref[...].astype(BF16),
                          preferred_element_type=F32)
        out_ref[...] = partial
        comm_ref[0, :, :] = partial.astype(BF16)

        for hop in range(N_DEV - 1):
            rdma = pltpu.make_async_remote_copy(
                src_ref=comm_ref.at[hop],
                dst_ref=comm_ref.at[hop + 1],
                send_sem=send_sems.at[hop],
                recv_sem=recv_sems.at[hop],
                device_id=(right,),
                device_id_type=pl.DeviceIdType.MESH,
            )
            rdma.start()
            rdma.wait()
            out_ref[...] += comm_ref[hop + 1, :, :].astype(F32)

    out = pl.pallas_call(
        body,
        out_shape=jax.ShapeDtypeStruct((SQ, DMODEL), F32),
        in_specs=[
            pl.BlockSpec(memory_space=pltpu.VMEM),
            pl.BlockSpec(memory_space=pltpu.VMEM),
            pl.BlockSpec(memory_space=pltpu.ANY),
            pl.BlockSpec(memory_space=pltpu.ANY),
            pl.BlockSpec(memory_space=pltpu.VMEM),
        ],
        out_specs=pl.BlockSpec(memory_space=pltpu.VMEM),
        scratch_shapes=[
            pltpu.VMEM((HQ_PER, NGRP, 4, QBLK, DH), F32),
            pltpu.VMEM((HQ_PER, NGRP, 4, QBLK, DH), F32),
            pltpu.VMEM((SQ, DMODEL), F32),
            pltpu.VMEM((N_DEV, SQ, DMODEL), BF16),
            pltpu.SemaphoreType.DMA((2,)),
            pltpu.SemaphoreType.DMA((N_DEV - 1,)),
            pltpu.SemaphoreType.DMA((N_DEV - 1,)),
        ],
        compiler_params=pltpu.CompilerParams(collective_id=0),
    )(x2, Wq, Kr, Vr, Wo)
    return out.reshape(1, SQ, DMODEL)


# baseline (device time: 50709 ns/iter reference)
import jax
import jax.numpy as jnp
from jax import lax
from jax.experimental import pallas as pl
from jax.experimental.pallas import tpu as pltpu

N_DEV = 4
HQ = 32
HQ_PER = 8
DH = 128
SQ = 256
SKV = 4096
DMODEL = 1024
QBLK = 64
NQB = SQ // QBLK
NGRP = SKV // (4 * QBLK)
SCALE = 0.08838834764831843
BF16 = jnp.bfloat16
F32 = jnp.float32


def kernel(x, Wq, K_ext, V_ext, Wo):
    x2 = x.reshape(SQ, DMODEL)
    Kr = K_ext.reshape(NGRP, 4, QBLK, HQ, DH)
    Vr = V_ext.reshape(NGRP, 4, QBLK, HQ, DH)

    def body(x_ref, wq_ref, k_hbm, v_hbm, wo_ref, out_ref,
             k_buf, v_buf, ctx_ref, comm_ref, copy_sems, send_sems, recv_sems):
        my = lax.axis_index("i")
        left = lax.rem(my + N_DEV - 1, N_DEV)
        right = lax.rem(my + 1, N_DEV)
        h0 = my * HQ_PER

        barrier = pltpu.get_barrier_semaphore()
        for nbr in (left, right):
            pl.semaphore_signal(barrier, inc=1, device_id=(nbr,),
                                device_id_type=pl.DeviceIdType.MESH)

        def start_color(qb, slot):
            descs = []
            for h in range(HQ_PER):
                ck = pltpu.make_async_copy(
                    k_hbm.at[:, qb, :, h0 + h, :], k_buf.at[slot, h],
                    copy_sems.at[slot, 0])
                cv = pltpu.make_async_copy(
                    v_hbm.at[:, qb, :, h0 + h, :], v_buf.at[slot, h],
                    copy_sems.at[slot, 1])
                ck.start()
                cv.start()
                descs += [ck, cv]
            return descs

        inflight = {0: start_color(0, 0)}

        q = lax.dot(x_ref[...].astype(BF16), wq_ref[...].astype(BF16),
                    preferred_element_type=F32)

        pl.semaphore_wait(barrier, 2)

        for qb in range(NQB):
            slot = qb % 2
            if qb + 1 < NQB:
                inflight[(qb + 1) % 2] = start_color(qb + 1, (qb + 1) % 2)
            for d in inflight[slot]:
                d.wait()
            for h in range(HQ_PER):
                q_blk = q[qb * QBLK:(qb + 1) * QBLK,
                          h * DH:(h + 1) * DH].astype(BF16)
                k_sel = k_buf[slot, h].reshape(NGRP * QBLK, DH)
                s = lax.dot_general(
                    q_blk, k_sel.astype(BF16), (((1,), (1,)), ((), ())),
                    preferred_element_type=F32) * SCALE
                m = jnp.max(s, axis=-1, keepdims=True)
                w = jnp.exp(s - m)
                p = (w / jnp.sum(w, axis=-1, keepdims=True)).astype(BF16)
                v_sel = v_buf[slot, h].reshape(NGRP * QBLK, DH)
                ctx_blk = lax.dot_general(
                    p, v_sel.astype(BF16), (((1,), (0,)), ((), ())),
                    preferred_element_type=F32)
                ctx_ref[qb * QBLK:(qb + 1) * QBLK,
                        h * DH:(h + 1) * DH] = ctx_blk

        partial = lax.dot(ctx_ref[...].astype(BF16), wo_ref[...].astype(BF16),
                          preferred_element_type=F32)
        out_ref[...] = partial
        comm_ref[0, :, :] = partial.astype(BF16)

        for hop in range(N_DEV - 1):
            rdma = pltpu.make_async_remote_copy(
                src_ref=comm_ref.at[hop],
                dst_ref=comm_ref.at[hop + 1],
                send_sem=send_sems.at[hop],
                recv_sem=recv_sems.at[hop],
                device_id=(right,),
                device_id_type=pl.DeviceIdType.MESH,
            )
            rdma.start()
            rdma.wait()
            out_ref[...] += comm_ref[hop + 1, :, :].astype(F32)

    out = pl.pallas_call(
        body,
        out_shape=jax.ShapeDtypeStruct((SQ, DMODEL), F32),
        in_specs=[
            pl.BlockSpec(memory_space=pltpu.VMEM),
            pl.BlockSpec(memory_space=pltpu.VMEM),
            pl.BlockSpec(memory_space=pl.ANY),
            pl.BlockSpec(memory_space=pl.ANY),
            pl.BlockSpec(memory_space=pltpu.VMEM),
        ],
        out_specs=pl.BlockSpec(memory_space=pltpu.VMEM),
        scratch_shapes=[
            pltpu.VMEM((2, HQ_PER, NGRP, QBLK, DH), F32),
            pltpu.VMEM((2, HQ_PER, NGRP, QBLK, DH), F32),
            pltpu.VMEM((SQ, DMODEL), F32),
            pltpu.VMEM((N_DEV, SQ, DMODEL), BF16),
            pltpu.SemaphoreType.DMA((2, 2)),
            pltpu.SemaphoreType.DMA((N_DEV - 1,)),
            pltpu.SemaphoreType.DMA((N_DEV - 1,)),
        ],
        compiler_params=pltpu.CompilerParams(collective_id=0),
    )(x2, Wq, Kr, Vr, Wo)
    return out.reshape(1, SQ, DMODEL)


# device time: 35149 ns/iter; 1.4427x vs baseline; 1.4427x over previous
import jax
import jax.numpy as jnp
from jax import lax
from jax.experimental import pallas as pl
from jax.experimental.pallas import tpu as pltpu

N_DEV = 4
HQ = 32
HQ_PER = 8
DH = 128
SQ = 256
SKV = 4096
DMODEL = 1024
QBLK = 64
NQB = SQ // QBLK
NGRP = SKV // (4 * QBLK)
SCALE = 0.08838834764831843
BF16 = jnp.bfloat16
F32 = jnp.float32


def kernel(x, Wq, K_ext, V_ext, Wo):
    x2 = x.reshape(SQ, DMODEL)
    Kr = K_ext.reshape(NGRP, 4, QBLK, HQ, DH)
    Vr = V_ext.reshape(NGRP, 4, QBLK, HQ, DH)

    def body(x_ref, wq_ref, k_hbm, v_hbm, wo_ref, out_ref,
             k_buf, v_buf, ctx_ref, comm_out, comm_in,
             copy_sems, send_sems, recv_sems):
        my = lax.axis_index("i")
        h0 = my * HQ_PER

        barrier = pltpu.get_barrier_semaphore()
        for d in range(1, N_DEV):
            pl.semaphore_signal(
                barrier, inc=1, device_id=(lax.rem(my + d, N_DEV),),
                device_id_type=pl.DeviceIdType.MESH)

        def start_color(qb, slot):
            descs = []
            for h in range(HQ_PER):
                ck = pltpu.make_async_copy(
                    k_hbm.at[:, qb, :, h0 + h, :], k_buf.at[slot, h],
                    copy_sems.at[slot, 0])
                cv = pltpu.make_async_copy(
                    v_hbm.at[:, qb, :, h0 + h, :], v_buf.at[slot, h],
                    copy_sems.at[slot, 1])
                ck.start()
                cv.start()
                descs += [ck, cv]
            return descs

        inflight = {0: start_color(0, 0)}

        q = lax.dot(x_ref[...].astype(BF16), wq_ref[...].astype(BF16),
                    preferred_element_type=F32)

        pl.semaphore_wait(barrier, N_DEV - 1)

        sends = []

        def broadcast_block(b):
            for d in range(1, N_DEV):
                tgt = lax.rem(my + d, N_DEV)
                rel = N_DEV - d - 1
                rdma = pltpu.make_async_remote_copy(
                    src_ref=comm_out.at[b],
                    dst_ref=comm_in.at[b, rel],
                    send_sem=send_sems.at[b, d - 1],
                    recv_sem=recv_sems.at[b, rel],
                    device_id=(tgt,),
                    device_id_type=pl.DeviceIdType.MESH,
                )
                rdma.start()
                sends.append(rdma)

        def drain_block(b):
            for rel in range(N_DEV - 1):
                recv = pltpu.make_async_remote_copy(
                    src_ref=comm_out.at[b],
                    dst_ref=comm_in.at[b, rel],
                    send_sem=send_sems.at[b, rel],
                    recv_sem=recv_sems.at[b, rel],
                    device_id=(my,),
                    device_id_type=pl.DeviceIdType.MESH,
                )
                recv.wait_recv()
            rows = pl.ds(b * QBLK, QBLK)
            acc = out_ref[rows, :]
            for rel in range(N_DEV - 1):
                acc = acc + comm_in[b, rel, :, :].astype(F32)
            out_ref[rows, :] = acc

        for qb in range(NQB):
            slot = qb % 2
            if qb + 1 < NQB:
                inflight[(qb + 1) % 2] = start_color(qb + 1, (qb + 1) % 2)
            for d in inflight[slot]:
                d.wait()
            for h in range(HQ_PER):
                q_blk = q[qb * QBLK:(qb + 1) * QBLK,
                          h * DH:(h + 1) * DH].astype(BF16)
                k_sel = k_buf[slot, h].reshape(NGRP * QBLK, DH)
                s = lax.dot_general(
                    q_blk, k_sel.astype(BF16), (((1,), (1,)), ((), ())),
                    preferred_element_type=F32) * SCALE
                m = jnp.max(s, axis=-1, keepdims=True)
                w = jnp.exp(s - m)
                p = (w / jnp.sum(w, axis=-1, keepdims=True)).astype(BF16)
                v_sel = v_buf[slot, h].reshape(NGRP * QBLK, DH)
                ctx_blk = lax.dot_general(
                    p, v_sel.astype(BF16), (((1,), (0,)), ((), ())),
                    preferred_element_type=F32)
                ctx_ref[qb * QBLK:(qb + 1) * QBLK,
                        h * DH:(h + 1) * DH] = ctx_blk

            rows = pl.ds(qb * QBLK, QBLK)
            partial = lax.dot(
                ctx_ref[rows, :].astype(BF16), wo_ref[...].astype(BF16),
                preferred_element_type=F32)
            out_ref[rows, :] = partial
            comm_out[qb, :, :] = partial.astype(BF16)
            broadcast_block(qb)
            if qb > 0:
                drain_block(qb - 1)

        drain_block(NQB - 1)
        for rdma in sends:
            rdma.wait_send()

    out = pl.pallas_call(
        body,
        out_shape=jax.ShapeDtypeStruct((SQ, DMODEL), F32),
        in_specs=[
            pl.BlockSpec(memory_space=pltpu.VMEM),
            pl.BlockSpec(memory_space=pltpu.VMEM),
            pl.BlockSpec(memory_space=pl.ANY),
            pl.BlockSpec(memory_space=pl.ANY),
            pl.BlockSpec(memory_space=pltpu.VMEM),
        ],
        out_specs=pl.BlockSpec(memory_space=pltpu.VMEM),
        scratch_shapes=[
            pltpu.VMEM((2, HQ_PER, NGRP, QBLK, DH), F32),
            pltpu.VMEM((2, HQ_PER, NGRP, QBLK, DH), F32),
            pltpu.VMEM((SQ, DMODEL), F32),
            pltpu.VMEM((NQB, QBLK, DMODEL), BF16),
            pltpu.VMEM((NQB, N_DEV - 1, QBLK, DMODEL), BF16),
            pltpu.SemaphoreType.DMA((2, 2)),
            pltpu.SemaphoreType.DMA((NQB, N_DEV - 1)),
            pltpu.SemaphoreType.DMA((NQB, N_DEV - 1)),
        ],
        compiler_params=pltpu.CompilerParams(collective_id=0),
    )(x2, Wq, Kr, Vr, Wo)
    return out.reshape(1, SQ, DMODEL)


# device time: 28583 ns/iter; 1.7741x vs baseline; 1.2297x over previous
import jax
import jax.numpy as jnp
from jax import lax
from jax.experimental import pallas as pl
from jax.experimental.pallas import tpu as pltpu

N_DEV = 4
HQ = 32
HQ_PER = 8
DH = 128
SQ = 256
SKV = 4096
DMODEL = 1024
QBLK = 64
NQB = SQ // QBLK
NGRP = SKV // (4 * QBLK)
NSLOT = 3
SCALE = 0.08838834764831843
BF16 = jnp.bfloat16
F32 = jnp.float32


def kernel(x, Wq, K_ext, V_ext, Wo):
    x2 = x.reshape(SQ, DMODEL)
    Kr = K_ext.reshape(NGRP, 4, QBLK, HQ, DH)
    Vr = V_ext.reshape(NGRP, 4, QBLK, HQ, DH)

    def body(x_ref, wq_ref, k_hbm, v_hbm, wo_ref, out_ref,
             k_buf, v_buf, ctx_ref, comm_out, comm_in,
             copy_sems, send_sems, recv_sems):
        my = lax.axis_index("i")
        h0 = my * HQ_PER

        barrier = pltpu.get_barrier_semaphore()
        for d in range(1, N_DEV):
            pl.semaphore_signal(
                barrier, inc=1, device_id=(lax.rem(my + d, N_DEV),),
                device_id_type=pl.DeviceIdType.MESH)

        def start_color(qb, slot):
            descs = []
            for h in range(HQ_PER):
                ck = pltpu.make_async_copy(
                    k_hbm.at[:, qb, :, h0 + h, :], k_buf.at[slot, h],
                    copy_sems.at[slot, 0])
                cv = pltpu.make_async_copy(
                    v_hbm.at[:, qb, :, h0 + h, :], v_buf.at[slot, h],
                    copy_sems.at[slot, 1])
                ck.start()
                cv.start()
                descs += [ck, cv]
            return descs

        inflight = {0: start_color(0, 0), 1: start_color(1, 1)}

        q = lax.dot(x_ref[...].astype(BF16), wq_ref[...].astype(BF16),
                    preferred_element_type=F32)

        pl.semaphore_wait(barrier, N_DEV - 1)

        sends = []

        def broadcast_block(b):
            for d in range(1, N_DEV):
                tgt = lax.rem(my + d, N_DEV)
                rel = N_DEV - d - 1
                rdma = pltpu.make_async_remote_copy(
                    src_ref=comm_out.at[b],
                    dst_ref=comm_in.at[b, rel],
                    send_sem=send_sems.at[b, d - 1],
                    recv_sem=recv_sems.at[b, rel],
                    device_id=(tgt,),
                    device_id_type=pl.DeviceIdType.MESH,
                )
                rdma.start()
                sends.append(rdma)

        def drain_block(b):
            for rel in range(N_DEV - 1):
                recv = pltpu.make_async_remote_copy(
                    src_ref=comm_out.at[b],
                    dst_ref=comm_in.at[b, rel],
                    send_sem=send_sems.at[b, rel],
                    recv_sem=recv_sems.at[b, rel],
                    device_id=(my,),
                    device_id_type=pl.DeviceIdType.MESH,
                )
                recv.wait_recv()
            rows = pl.ds(b * QBLK, QBLK)
            acc = out_ref[rows, :]
            for rel in range(N_DEV - 1):
                acc = acc + comm_in[b, rel, :, :].astype(F32)
            out_ref[rows, :] = acc

        for qb in range(NQB):
            slot = qb % NSLOT
            if qb + 2 < NQB:
                inflight[(qb + 2) % NSLOT] = start_color(
                    qb + 2, (qb + 2) % NSLOT)
            for dsc in inflight[slot]:
                dsc.wait()
            for h in range(HQ_PER):
                q_blk = q[qb * QBLK:(qb + 1) * QBLK,
                          h * DH:(h + 1) * DH].astype(BF16)
                k_sel = k_buf[slot, h].reshape(NGRP * QBLK, DH)
                s = lax.dot_general(
                    q_blk, k_sel.astype(BF16), (((1,), (1,)), ((), ())),
                    preferred_element_type=F32) * SCALE
                w = jnp.exp(s)
                wsum = jnp.sum(w, axis=-1, keepdims=True)
                v_sel = v_buf[slot, h].reshape(NGRP * QBLK, DH)
                ctx_blk = lax.dot_general(
                    w.astype(BF16), v_sel.astype(BF16),
                    (((1,), (0,)), ((), ())),
                    preferred_element_type=F32)
                ctx_ref[qb * QBLK:(qb + 1) * QBLK,
                        h * DH:(h + 1) * DH] = ctx_blk / wsum

            rows = pl.ds(qb * QBLK, QBLK)
            partial = lax.dot(
                ctx_ref[rows, :].astype(BF16), wo_ref[...].astype(BF16),
                preferred_element_type=F32)
            out_ref[rows, :] = partial
            comm_out[qb, :, :] = partial.astype(BF16)
            broadcast_block(qb)
            if qb >= 2:
                drain_block(qb - 2)

        drain_block(NQB - 2)
        drain_block(NQB - 1)
        for rdma in sends:
            rdma.wait_send()

    out = pl.pallas_call(
        body,
        out_shape=jax.ShapeDtypeStruct((SQ, DMODEL), F32),
        in_specs=[
            pl.BlockSpec(memory_space=pltpu.VMEM),
            pl.BlockSpec(memory_space=pltpu.VMEM),
            pl.BlockSpec(memory_space=pl.ANY),
            pl.BlockSpec(memory_space=pl.ANY),
            pl.BlockSpec(memory_space=pltpu.VMEM),
        ],
        out_specs=pl.BlockSpec(memory_space=pltpu.VMEM),
        scratch_shapes=[
            pltpu.VMEM((NSLOT, HQ_PER, NGRP, QBLK, DH), F32),
            pltpu.VMEM((NSLOT, HQ_PER, NGRP, QBLK, DH), F32),
            pltpu.VMEM((SQ, DMODEL), F32),
            pltpu.VMEM((NQB, QBLK, DMODEL), BF16),
            pltpu.VMEM((NQB, N_DEV - 1, QBLK, DMODEL), BF16),
            pltpu.SemaphoreType.DMA((NSLOT, 2)),
            pltpu.SemaphoreType.DMA((NQB, N_DEV - 1)),
            pltpu.SemaphoreType.DMA((NQB, N_DEV - 1)),
        ],
        compiler_params=pltpu.CompilerParams(collective_id=0),
    )(x2, Wq, Kr, Vr, Wo)
    return out.reshape(1, SQ, DMODEL)


# device time: 28516 ns/iter; 1.7783x vs baseline; 1.0023x over previous
import jax
import jax.numpy as jnp
from jax import lax
from jax.experimental import pallas as pl
from jax.experimental.pallas import tpu as pltpu

N_DEV = 4
HQ = 32
HQ_PER = 8
DH = 128
SQ = 256
SKV = 4096
DMODEL = 1024
QBLK = 64
NQB = SQ // QBLK
NGRP = SKV // (4 * QBLK)
NSLOT = 3
SCALE = 0.08838834764831843
BF16 = jnp.bfloat16
F32 = jnp.float32


def kernel(x, Wq, K_ext, V_ext, Wo):
    x2 = x.reshape(SQ, DMODEL)
    Kr = K_ext.reshape(NGRP, 4, QBLK, HQ, DH)
    Vr = V_ext.reshape(NGRP, 4, QBLK, HQ, DH)

    def body(x_ref, wq_ref, k_hbm, v_hbm, wo_ref, out_ref,
             k_buf, v_buf, ctx_ref, comm_out, comm_in,
             copy_sems, send_sems, recv_sems):
        my = lax.axis_index("i")
        h0 = my * HQ_PER

        barrier = pltpu.get_barrier_semaphore()
        for d in range(1, N_DEV):
            pl.semaphore_signal(
                barrier, inc=1, device_id=(lax.rem(my + d, N_DEV),),
                device_id_type=pl.DeviceIdType.MESH)

        def start_color(qb, slot):
            descs = []
            for h in range(HQ_PER):
                ck = pltpu.make_async_copy(
                    k_hbm.at[:, qb, :, h0 + h, :], k_buf.at[slot, h],
                    copy_sems.at[slot, 0])
                cv = pltpu.make_async_copy(
                    v_hbm.at[:, qb, :, h0 + h, :], v_buf.at[slot, h],
                    copy_sems.at[slot, 1])
                ck.start()
                cv.start()
                descs += [ck, cv]
            return descs

        inflight = {0: start_color(0, 0), 1: start_color(1, 1)}

        q = lax.dot(x_ref[...].astype(BF16), wq_ref[...].astype(BF16),
                    preferred_element_type=F32)

        pl.semaphore_wait(barrier, N_DEV - 1)

        sends = []

        def broadcast_block(b):
            for d in (2, 1, 3):
                tgt = lax.rem(my + d, N_DEV)
                rel = N_DEV - d - 1
                rdma = pltpu.make_async_remote_copy(
                    src_ref=comm_out.at[b],
                    dst_ref=comm_in.at[b, rel],
                    send_sem=send_sems.at[b, d - 1],
                    recv_sem=recv_sems.at[b, rel],
                    device_id=(tgt,),
                    device_id_type=pl.DeviceIdType.MESH,
                )
                rdma.start()
                sends.append(rdma)

        def drain_block(b):
            for rel in range(N_DEV - 1):
                recv = pltpu.make_async_remote_copy(
                    src_ref=comm_out.at[b],
                    dst_ref=comm_in.at[b, rel],
                    send_sem=send_sems.at[b, rel],
                    recv_sem=recv_sems.at[b, rel],
                    device_id=(my,),
                    device_id_type=pl.DeviceIdType.MESH,
                )
                recv.wait_recv()
            rows = pl.ds(b * QBLK, QBLK)
            acc = out_ref[rows, :]
            for rel in range(N_DEV - 1):
                acc = acc + comm_in[b, rel, :, :].astype(F32)
            out_ref[rows, :] = acc

        for qb in range(NQB):
            slot = qb % NSLOT
            if qb + 2 < NQB:
                inflight[(qb + 2) % NSLOT] = start_color(
                    qb + 2, (qb + 2) % NSLOT)
            for dsc in inflight[slot]:
                dsc.wait()
            for h in range(HQ_PER):
                q_blk = q[qb * QBLK:(qb + 1) * QBLK,
                          h * DH:(h + 1) * DH].astype(BF16)
                k_sel = k_buf[slot, h].reshape(NGRP * QBLK, DH)
                s = lax.dot_general(
                    q_blk, k_sel.astype(BF16), (((1,), (1,)), ((), ())),
                    preferred_element_type=F32) * SCALE
                w = jnp.exp(s)
                wsum = jnp.sum(w, axis=-1, keepdims=True)
                v_sel = v_buf[slot, h].reshape(NGRP * QBLK, DH)
                ctx_blk = lax.dot_general(
                    w.astype(BF16), v_sel.astype(BF16),
                    (((1,), (0,)), ((), ())),
                    preferred_element_type=F32)
                ctx_ref[qb * QBLK:(qb + 1) * QBLK,
                        h * DH:(h + 1) * DH] = ctx_blk / wsum

            rows = pl.ds(qb * QBLK, QBLK)
            partial = lax.dot(
                ctx_ref[rows, :].astype(BF16), wo_ref[...].astype(BF16),
                preferred_element_type=F32)
            out_ref[rows, :] = partial
            comm_out[qb, :, :] = partial.astype(BF16)
            broadcast_block(qb)
            if qb >= 2:
                drain_block(qb - 2)

        drain_block(NQB - 2)
        drain_block(NQB - 1)
        for rdma in sends:
            rdma.wait_send()

    out = pl.pallas_call(
        body,
        out_shape=jax.ShapeDtypeStruct((SQ, DMODEL), F32),
        in_specs=[
            pl.BlockSpec(memory_space=pltpu.VMEM),
            pl.BlockSpec(memory_space=pltpu.VMEM),
            pl.BlockSpec(memory_space=pl.ANY),
            pl.BlockSpec(memory_space=pl.ANY),
            pl.BlockSpec(memory_space=pltpu.VMEM),
        ],
        out_specs=pl.BlockSpec(memory_space=pltpu.VMEM),
        scratch_shapes=[
            pltpu.VMEM((NSLOT, HQ_PER, NGRP, QBLK, DH), F32),
            pltpu.VMEM((NSLOT, HQ_PER, NGRP, QBLK, DH), F32),
            pltpu.VMEM((SQ, DMODEL), F32),
            pltpu.VMEM((NQB, QBLK, DMODEL), BF16),
            pltpu.VMEM((NQB, N_DEV - 1, QBLK, DMODEL), BF16),
            pltpu.SemaphoreType.DMA((NSLOT, 2)),
            pltpu.SemaphoreType.DMA((NQB, N_DEV - 1)),
            pltpu.SemaphoreType.DMA((NQB, N_DEV - 1)),
        ],
        compiler_params=pltpu.CompilerParams(collective_id=0),
    )(x2, Wq, Kr, Vr, Wo)
    return out.reshape(1, SQ, DMODEL)
